# baseline (speedup 1.0000x reference)
import functools
import numpy as np
import jax
import jax.numpy as jnp
from jax import lax
from jax.experimental import pallas as pl
from jax.experimental.pallas import tpu as pltpu
from jax.experimental.pallas import tpu_sc as plsc

F32 = jnp.float32
I32 = jnp.int32


def _fps_body(P, xs, ys, zs, cx_o, cy_o, cz_o):
    B, N = xs.shape
    iota = lax.broadcasted_iota(I32, (B, N), 1)
    x = xs[:, :]
    y = ys[:, :]
    z = zs[:, :]

    piota = lax.broadcasted_iota(I32, (B, P), 1)

    def body(i, c):
        dist, far, cxa, cya, cza = c
        oh = iota == far
        cx = jnp.sum(jnp.where(oh, x, 0.0), axis=-1, keepdims=True)
        cy = jnp.sum(jnp.where(oh, y, 0.0), axis=-1, keepdims=True)
        cz = jnp.sum(jnp.where(oh, z, 0.0), axis=-1, keepdims=True)
        sel = piota == i
        cxa = jnp.where(sel, cx, cxa)
        cya = jnp.where(sel, cy, cya)
        cza = jnp.where(sel, cz, cza)
        dx = x - cx
        dy = y - cy
        dz = z - cz
        d = dx * dx + dy * dy + dz * dz
        dist = jnp.minimum(dist, d)
        m = jnp.max(dist, axis=-1, keepdims=True)
        far = jnp.min(jnp.where(dist == m, iota, N), axis=-1, keepdims=True)
        return dist, far, cxa, cya, cza

    dist0 = jnp.full((B, N), 1e10, dtype=F32)
    far0 = jnp.zeros((B, 1), dtype=I32)
    z0 = jnp.zeros((B, P), dtype=F32)
    _, _, cxa, cya, cza = lax.fori_loop(0, P, body, (dist0, far0, z0, z0, z0))
    cx_o[:, :] = cxa
    cy_o[:, :] = cya
    cz_o[:, :] = cza


def _fps(xs, ys, zs, P):
    B, N = xs.shape
    out = [jax.ShapeDtypeStruct((B, P), F32)] * 3
    return pl.pallas_call(
        functools.partial(_fps_body, P),
        out_shape=out,
    )(xs, ys, zs)


def _bq_body(r2, nsample, N, flat, qx, qy, qz, bx, by, bz, out):
    SB = qx.shape[1]
    dx = qx[0] - bx[0]
    dy = qy[0] - by[0]
    dz = qz[0] - bz[0]
    sqr = dx * dx + dy * dy + dz * dz
    iota = lax.broadcasted_iota(I32, (SB, N), 1)
    cur = jnp.where(sqr <= r2, iota, N)
    base = pl.program_id(0) * N if flat else 0
    kiota = lax.broadcasted_iota(I32, (SB, nsample), 1)

    slot0 = jnp.min(cur, axis=-1, keepdims=True)
    cur = jnp.where(cur == slot0, N, cur)
    slots0 = jnp.broadcast_to(slot0, (SB, nsample))
    mk1 = jnp.min(cur, axis=-1, keepdims=True)

    def cond(c):
        k, _, _, mk = c
        return jnp.logical_and(k < nsample, jnp.min(mk) < N)

    def body(c):
        k, cur, slots, mk = c
        hit = jnp.logical_and(kiota == k, mk < N)
        slots = jnp.where(hit, mk, slots)
        cur = jnp.where(cur == mk, N, cur)
        mk = jnp.min(cur, axis=-1, keepdims=True)
        return k + 1, cur, slots, mk

    _, _, slots, _ = lax.while_loop(cond, body, (1, cur, slots0, mk1))
    out[0, :, :nsample] = slots + base


def _ball_query(radius, nsample, bx, by, bz, qxc, qyc, qzc, flat, s_blk):
    B, N = bx.shape
    S = qxc.shape[1]
    r2 = float(np.float32(radius * radius))
    grid = (B, S // s_blk)
    b3 = lambda a: a[:, None, :]
    return pl.pallas_call(
        functools.partial(_bq_body, r2, nsample, N, flat),
        grid=grid,
        compiler_params=pltpu.CompilerParams(
            dimension_semantics=("parallel", "parallel")),
        in_specs=[
            pl.BlockSpec((1, s_blk, 1), lambda b, s: (b, s, 0)),
            pl.BlockSpec((1, s_blk, 1), lambda b, s: (b, s, 0)),
            pl.BlockSpec((1, s_blk, 1), lambda b, s: (b, s, 0)),
            pl.BlockSpec((1, 1, N), lambda b, s: (b, 0, 0)),
            pl.BlockSpec((1, 1, N), lambda b, s: (b, 0, 0)),
            pl.BlockSpec((1, 1, N), lambda b, s: (b, 0, 0)),
        ],
        out_specs=pl.BlockSpec((1, s_blk, 128), lambda b, s: (b, s, 0)),
        out_shape=jax.ShapeDtypeStruct((B, S, 128), I32),
    )(qxc, qyc, qzc, b3(bx), b3(by), b3(bz))


def _mlp1_body(x, w1, b1, w2, b2, w3, b3, out):
    h = jnp.maximum(jnp.dot(x[:, :], w1[:, :], preferred_element_type=F32) + b1[:, :], 0.0)
    h = jnp.maximum(jnp.dot(h, w2[:, :], preferred_element_type=F32) + b2[:, :], 0.0)
    h = jnp.maximum(jnp.dot(h, w3[:, :], preferred_element_type=F32) + b3[:, :], 0.0)
    out[:, :] = h


def _mlp1(x8, w1p, b1, w2, b2, w3, b3, blk):
    R = x8.shape[0]
    grid = (R // blk,)
    return pl.pallas_call(
        _mlp1_body,
        grid=grid,
        compiler_params=pltpu.CompilerParams(
            dimension_semantics=("parallel",)),
        in_specs=[
            pl.BlockSpec((blk, 8), lambda i: (i, 0)),
            pl.BlockSpec(w1p.shape, lambda i: (0, 0)),
            pl.BlockSpec(b1.shape, lambda i: (0, 0)),
            pl.BlockSpec(w2.shape, lambda i: (0, 0)),
            pl.BlockSpec(b2.shape, lambda i: (0, 0)),
            pl.BlockSpec(w3.shape, lambda i: (0, 0)),
            pl.BlockSpec(b3.shape, lambda i: (0, 0)),
        ],
        out_specs=pl.BlockSpec((blk, 128), lambda i: (i, 0)),
        out_shape=jax.ShapeDtypeStruct((R, 128), F32),
    )(x8, w1p, b1, w2, b2, w3, b3)


def _gmax_body(nsample, N, gi, h, out):
    SB = gi.shape[1]
    g = gi[0][:, :nsample]
    hb = h[0]
    iota = lax.broadcasted_iota(I32, (SB, N), 1)
    acc = jnp.full((SB, hb.shape[1]), -jnp.inf, dtype=F32)
    for k in range(nsample):
        oh = (g[:, k:k + 1] == iota).astype(F32)
        acc = jnp.maximum(acc, jnp.dot(oh, hb, preferred_element_type=F32))
    out[0] = acc


def _gather_max(gi, h3, nsample, s_blk):
    B, S, _ = gi.shape
    N, C = h3.shape[1], h3.shape[2]
    grid = (B, S // s_blk)
    return pl.pallas_call(
        functools.partial(_gmax_body, nsample, N),
        grid=grid,
        compiler_params=pltpu.CompilerParams(
            dimension_semantics=("parallel", "parallel")),
        in_specs=[
            pl.BlockSpec((1, s_blk, 128), lambda b, s: (b, s, 0)),
            pl.BlockSpec((1, N, C), lambda b, s: (b, 0, 0)),
        ],
        out_specs=pl.BlockSpec((1, s_blk, C), lambda b, s: (b, s, 0)),
        out_shape=jax.ShapeDtypeStruct((B, S, C), F32),
    )(gi, h3)


def _sa2_body(nsample, N, gi, x1p, f1, c8, w1a, w1b, b1, w2, b2, w3, b3, out):
    S = gi.shape[1]
    g = gi[0][:, :nsample]
    x1 = x1p[0]
    f1b = f1[0]
    ctr = c8[0]
    iota = lax.broadcasted_iota(I32, (S, N), 1)
    acc = jnp.full((S, w3.shape[1]), -jnp.inf, dtype=F32)
    for k in range(nsample):
        oh = (g[:, k:k + 1] == iota).astype(F32)
        gx = jnp.dot(oh, x1, preferred_element_type=F32) - ctr
        gf = jnp.dot(oh, f1b, preferred_element_type=F32)
        h = jnp.maximum(jnp.dot(gx, w1a[:, :], preferred_element_type=F32)
                        + jnp.dot(gf, w1b[:, :], preferred_element_type=F32)
                        + b1[:, :], 0.0)
        h = jnp.maximum(jnp.dot(h, w2[:, :], preferred_element_type=F32) + b2[:, :], 0.0)
        h = jnp.maximum(jnp.dot(h, w3[:, :], preferred_element_type=F32) + b3[:, :], 0.0)
        acc = jnp.maximum(acc, h)
    out[0] = acc


def _sa2(gi2, x1p8, f1, c8, w1a, w1b, b1, w2, b2, w3, b3, nsample):
    B, S, _ = gi2.shape
    N = x1p8.shape[1]
    Cout = w3.shape[1]
    return pl.pallas_call(
        functools.partial(_sa2_body, nsample, N),
        grid=(B,),
        compiler_params=pltpu.CompilerParams(
            dimension_semantics=("parallel",)),
        in_specs=[
            pl.BlockSpec((1, S, 128), lambda b: (b, 0, 0)),
            pl.BlockSpec((1, N, 8), lambda b: (b, 0, 0)),
            pl.BlockSpec((1, N, 128), lambda b: (b, 0, 0)),
            pl.BlockSpec((1, S, 8), lambda b: (b, 0, 0)),
            pl.BlockSpec(w1a.shape, lambda b: (0, 0)),
            pl.BlockSpec(w1b.shape, lambda b: (0, 0)),
            pl.BlockSpec(b1.shape, lambda b: (0, 0)),
            pl.BlockSpec(w2.shape, lambda b: (0, 0)),
            pl.BlockSpec(b2.shape, lambda b: (0, 0)),
            pl.BlockSpec(w3.shape, lambda b: (0, 0)),
            pl.BlockSpec(b3.shape, lambda b: (0, 0)),
        ],
        out_specs=pl.BlockSpec((1, S, Cout), lambda b: (b, 0, 0)),
        out_shape=jax.ShapeDtypeStruct((B, S, Cout), F32),
    )(gi2, x1p8, f1, c8, w1a, w1b, b1, w2, b2, w3, b3)


def _sa3fp1_body(c8, f2, w3a, w3b, b1, w2, b2, w3, b3, wfa, wfb, bf1, wf2, bf2, out):
    x = c8[0]
    f = f2[0]
    h = jnp.maximum(jnp.dot(x, w3a[:, :], preferred_element_type=F32)
                    + jnp.dot(f, w3b[:, :], preferred_element_type=F32) + b1[:, :], 0.0)
    h = jnp.maximum(jnp.dot(h, w2[:, :], preferred_element_type=F32) + b2[:, :], 0.0)
    h = jnp.maximum(jnp.dot(h, w3[:, :], preferred_element_type=F32) + b3[:, :], 0.0)
    f3 = jnp.max(h, axis=0, keepdims=True)
    u = jnp.maximum(jnp.dot(f3, wfa[:, :], preferred_element_type=F32)
                    + jnp.dot(f, wfb[:, :], preferred_element_type=F32) + bf1[:, :], 0.0)
    u = jnp.maximum(jnp.dot(u, wf2[:, :], preferred_element_type=F32) + bf2[:, :], 0.0)
    out[0] = u


def _sa3fp1(c8, f2, w3a, w3b, b1, w2, b2, w3, b3, wfa, wfb, bf1, wf2, bf2):
    B, S, _ = f2.shape
    Cout = wf2.shape[1]
    full = lambda a: pl.BlockSpec(a.shape, lambda b: tuple(0 for _ in a.shape))
    return pl.pallas_call(
        _sa3fp1_body,
        grid=(B,),
        compiler_params=pltpu.CompilerParams(
            dimension_semantics=("parallel",)),
        in_specs=[
            pl.BlockSpec((1, S, 8), lambda b: (b, 0, 0)),
            pl.BlockSpec((1, S, 256), lambda b: (b, 0, 0)),
            full(w3a), full(w3b), full(b1), full(w2), full(b2), full(w3), full(b3),
            full(wfa), full(wfb), full(bf1), full(wf2), full(bf2),
        ],
        out_specs=pl.BlockSpec((1, S, Cout), lambda b: (b, 0, 0)),
        out_shape=jax.ShapeDtypeStruct((B, S, Cout), F32),
    )(c8, f2, w3a, w3b, b1, w2, b2, w3, b3, wfa, wfb, bf1, wf2, bf2)


def _fp2_body(M, qx, qy, qz, bx, by, bz, f2u, f1, wa, wb, b1, w2, b2, out):
    S = qx.shape[1]
    dx = qx[0] - bx[0]
    dy = qy[0] - by[0]
    dz = qz[0] - bz[0]
    sqr = dx * dx + dy * dy + dz * dz
    iota = lax.broadcasted_iota(I32, (S, M), 1)
    fu = f2u[0]
    ws = []
    gs = []
    for _ in range(3):
        mv = jnp.min(sqr, axis=-1, keepdims=True)
        am = jnp.min(jnp.where(sqr == mv, iota, M), axis=-1, keepdims=True)
        oh = (iota == am).astype(F32)
        gs.append(jnp.dot(oh, fu, preferred_element_type=F32))
        ws.append(1.0 / jnp.maximum(mv, 1e-10))
        sqr = jnp.where(iota == am, jnp.float32(1e30), sqr)
    wsum = ws[0] + ws[1] + ws[2]
    interp = (ws[0] * gs[0] + ws[1] * gs[1] + ws[2] * gs[2]) / wsum
    h = jnp.maximum(jnp.dot(interp, wa[:, :], preferred_element_type=F32)
                    + jnp.dot(f1[0], wb[:, :], preferred_element_type=F32) + b1[:, :], 0.0)
    out[0] = jnp.maximum(jnp.dot(h, w2[:, :], preferred_element_type=F32) + b2[:, :], 0.0)


def _fp2(qxc, qyc, qzc, bx, by, bz, f2u, f1, wa, wb, b1, w2, b2):
    B, S, _ = qxc.shape
    M = bx.shape[1]
    Cout = w2.shape[1]
    full = lambda a: pl.BlockSpec(a.shape, lambda b: tuple(0 for _ in a.shape))
    return pl.pallas_call(
        functools.partial(_fp2_body, M),
        grid=(B,),
        compiler_params=pltpu.CompilerParams(
            dimension_semantics=("parallel",)),
        in_specs=[
            pl.BlockSpec((1, S, 1), lambda b: (b, 0, 0)),
            pl.BlockSpec((1, S, 1), lambda b: (b, 0, 0)),
            pl.BlockSpec((1, S, 1), lambda b: (b, 0, 0)),
            pl.BlockSpec((1, 1, M), lambda b: (b, 0, 0)),
            pl.BlockSpec((1, 1, M), lambda b: (b, 0, 0)),
            pl.BlockSpec((1, 1, M), lambda b: (b, 0, 0)),
            pl.BlockSpec((1, M, 256), lambda b: (b, 0, 0)),
            pl.BlockSpec((1, S, 128), lambda b: (b, 0, 0)),
            full(wa), full(wb), full(b1), full(w2), full(b2),
        ],
        out_specs=pl.BlockSpec((1, S, Cout), lambda b: (b, 0, 0)),
        out_shape=jax.ShapeDtypeStruct((B, S, Cout), F32),
    )(qxc, qyc, qzc, bx[:, None, :], by[:, None, :], bz[:, None, :],
      f2u, f1, wa, wb, b1, w2, b2)


def _sc_gather_max(table, idx, nsample):
    info = plsc.get_sparse_core_info()
    NC, NS = info.num_cores, info.num_subcores
    NW = NC * NS
    M = idx.shape[0]
    per_w = M // NW
    CH = 256
    n_ch = per_w // CH
    OUT_CH = CH // nsample
    mesh = plsc.VectorSubcoreMesh(core_axis_name="c", subcore_axis_name="s")

    @functools.partial(
        pl.kernel, mesh=mesh,
        out_type=jax.ShapeDtypeStruct((M // nsample, 128), F32),
        scratch_types=[
            pltpu.VMEM((CH,), I32),
            pltpu.VMEM((CH,), I32),
            pltpu.VMEM((CH, 128), F32),
            pltpu.VMEM((CH, 128), F32),
            pltpu.VMEM((OUT_CH, 128), F32),
            pltpu.SemaphoreType.DMA,
            pltpu.SemaphoreType.DMA,
        ],
    )
    def k(table_hbm, idx_hbm, out_hbm, idx0, idx1, rows0, rows1, out_v,
          sem0, sem1):
        wid = lax.axis_index("s") * NC + lax.axis_index("c")
        idx_v = (idx0, idx1)
        rows_v = (rows0, rows1)
        sems = (sem0, sem1)

        def fire(ch):
            base = pl.multiple_of(wid * per_w + ch * CH, CH)
            pltpu.sync_copy(idx_hbm.at[pl.ds(base, CH)], idx_v[ch % 2])
            return pltpu.async_copy(
                table_hbm.at[idx_v[ch % 2]], rows_v[ch % 2], sems[ch % 2])

        copies = {0: fire(0)}
        for ch in range(n_ch):
            if ch + 1 < n_ch:
                copies[ch + 1] = fire(ch + 1)
            copies[ch].wait()
            rv = rows_v[ch % 2]

            def grp(g, _):
                def red(r, acc):
                    row = g * nsample + r
                    return tuple(
                        jnp.maximum(acc[v], rv[row, pl.ds(16 * v, 16)])
                        for v in range(8))

                acc = lax.fori_loop(
                    0, nsample, red,
                    tuple(jnp.zeros((16,), F32) for _ in range(8)))
                for v in range(8):
                    out_v[g, pl.ds(16 * v, 16)] = acc[v]
                return 0

            lax.fori_loop(0, OUT_CH, grp, 0)
            ob = pl.multiple_of((wid * per_w + ch * CH) // nsample, OUT_CH)
            pltpu.sync_copy(out_v, out_hbm.at[pl.ds(ob, OUT_CH)])

    return k(table, idx)


def _pad_rows(w, rows):
    return jnp.concatenate([w, jnp.zeros((rows - w.shape[0], w.shape[1]), F32)], axis=0)


def kernel(xyz, params):
    B, N, _ = xyz.shape
    xs = xyz[:, :, 0]
    ys = xyz[:, :, 1]
    zs = xyz[:, :, 2]

    cx1, cy1, cz1 = _fps(xs, ys, zs, 512)
    gi1 = _ball_query(0.1, 16, xs, ys, zs,
                      cx1[:, :, None], cy1[:, :, None], cz1[:, :, None],
                      flat=True, s_blk=64)
    w1, w2, w3 = params['sa1_w']
    b1, b2, b3 = params['sa1_b']
    x8 = jnp.concatenate([xyz, jnp.zeros((B, N, 5), F32)], axis=-1).reshape(B * N, 8)
    h_all = _mlp1(x8, _pad_rows(w1, 8), b1[None, :], w2, b2[None, :], w3, b3[None, :],
                  blk=4096)
    f1 = _sc_gather_max(h_all, gi1[:, :, :16].reshape(-1), 16).reshape(B, 512, 128)

    cx2, cy2, cz2 = _fps(cx1, cy1, cz1, 128)
    gi2 = _ball_query(0.2, 16, cx1, cy1, cz1,
                      cx2[:, :, None], cy2[:, :, None], cz2[:, :, None],
                      flat=False, s_blk=128)
    x1p8 = jnp.concatenate(
        [cx1[:, :, None], cy1[:, :, None], cz1[:, :, None],
         jnp.zeros((B, 512, 5), F32)], axis=-1)
    c8 = jnp.concatenate(
        [cx2[:, :, None], cy2[:, :, None], cz2[:, :, None],
         jnp.zeros((B, 128, 5), F32)], axis=-1)
    sw1, sw2, sw3 = params['sa2_w']
    sb1, sb2, sb3 = params['sa2_b']
    f2 = _sa2(gi2, x1p8, f1, c8,
              _pad_rows(sw1[:3], 8), sw1[3:], sb1[None, :],
              sw2, sb2[None, :], sw3, sb3[None, :], nsample=16)

    tw1, tw2, tw3 = params['sa3_w']
    tb1, tb2, tb3 = params['sa3_b']
    fw1, fw2 = params['fp1_w']
    fb1, fb2 = params['fp1_b']
    f2u = _sa3fp1(c8, f2,
                  _pad_rows(tw1[:3], 8), tw1[3:], tb1[None, :],
                  tw2, tb2[None, :], tw3, tb3[None, :],
                  fw1[:1024], fw1[1024:], fb1[None, :], fw2, fb2[None, :])

    pw1, pw2 = params['fp2_w']
    pb1, pb2 = params['fp2_b']
    out = _fp2(cx1[:, :, None], cy1[:, :, None], cz1[:, :, None],
               cx2, cy2, cz2, f2u, f1,
               pw1[:256], pw1[256:], pb1[None, :], pw2, pb2[None, :])
    return out


# speedup vs baseline: 1.0144x; 1.0144x over previous
import functools
import numpy as np
import jax
import jax.numpy as jnp
from jax import lax
from jax.experimental import pallas as pl
from jax.experimental.pallas import tpu as pltpu
from jax.experimental.pallas import tpu_sc as plsc

F32 = jnp.float32
I32 = jnp.int32


def _fps_body(P, xs, ys, zs, cx_o, cy_o, cz_o):
    B, N = xs.shape
    iota = lax.broadcasted_iota(I32, (B, N), 1)
    x = xs[:, :]
    y = ys[:, :]
    z = zs[:, :]

    piota = lax.broadcasted_iota(I32, (B, P), 1)

    def body(i, c):
        dist, far, cxa, cya, cza = c
        oh = iota == far
        cx = jnp.sum(jnp.where(oh, x, 0.0), axis=-1, keepdims=True)
        cy = jnp.sum(jnp.where(oh, y, 0.0), axis=-1, keepdims=True)
        cz = jnp.sum(jnp.where(oh, z, 0.0), axis=-1, keepdims=True)
        sel = piota == i
        cxa = jnp.where(sel, cx, cxa)
        cya = jnp.where(sel, cy, cya)
        cza = jnp.where(sel, cz, cza)
        dx = x - cx
        dy = y - cy
        dz = z - cz
        d = dx * dx + dy * dy + dz * dz
        dist = jnp.minimum(dist, d)
        m = jnp.max(dist, axis=-1, keepdims=True)
        far = jnp.min(jnp.where(dist == m, iota, N), axis=-1, keepdims=True)
        return dist, far, cxa, cya, cza

    dist0 = jnp.full((B, N), 1e10, dtype=F32)
    far0 = jnp.zeros((B, 1), dtype=I32)
    z0 = jnp.zeros((B, P), dtype=F32)
    _, _, cxa, cya, cza = lax.fori_loop(0, P, body, (dist0, far0, z0, z0, z0))
    cx_o[:, :] = cxa
    cy_o[:, :] = cya
    cz_o[:, :] = cza


def _fps(xs, ys, zs, P):
    B, N = xs.shape
    out = [jax.ShapeDtypeStruct((B, P), F32)] * 3
    return pl.pallas_call(
        functools.partial(_fps_body, P),
        out_shape=out,
    )(xs, ys, zs)


def _bq_body(r2, nsample, N, flat, qx, qy, qz, bx, by, bz, out):
    SB = qx.shape[1]
    dx = qx[0] - bx[0]
    dy = qy[0] - by[0]
    dz = qz[0] - bz[0]
    sqr = dx * dx + dy * dy + dz * dz
    iota = lax.broadcasted_iota(I32, (SB, N), 1)
    cur = jnp.where(sqr <= r2, iota, N)
    base = pl.program_id(0) * N if flat else 0
    kiota = lax.broadcasted_iota(I32, (SB, nsample), 1)

    slot0 = jnp.min(cur, axis=-1, keepdims=True)
    cur = jnp.where(cur == slot0, N, cur)
    slots0 = jnp.broadcast_to(slot0, (SB, nsample))
    mk1 = jnp.min(cur, axis=-1, keepdims=True)

    def cond(c):
        k, _, _, mk = c
        return jnp.logical_and(k < nsample, jnp.min(mk) < N)

    def body(c):
        k, cur, slots, mk = c
        hit = jnp.logical_and(kiota == k, mk < N)
        slots = jnp.where(hit, mk, slots)
        cur = jnp.where(cur == mk, N, cur)
        mk = jnp.min(cur, axis=-1, keepdims=True)
        return k + 1, cur, slots, mk

    _, _, slots, _ = lax.while_loop(cond, body, (1, cur, slots0, mk1))
    out[0, :, :nsample] = slots + base


def _ball_query(radius, nsample, bx, by, bz, qxc, qyc, qzc, flat, s_blk):
    B, N = bx.shape
    S = qxc.shape[1]
    r2 = float(np.float32(radius * radius))
    grid = (B, S // s_blk)
    b3 = lambda a: a[:, None, :]
    return pl.pallas_call(
        functools.partial(_bq_body, r2, nsample, N, flat),
        grid=grid,
        compiler_params=pltpu.CompilerParams(
            dimension_semantics=("parallel", "parallel")),
        in_specs=[
            pl.BlockSpec((1, s_blk, 1), lambda b, s: (b, s, 0)),
            pl.BlockSpec((1, s_blk, 1), lambda b, s: (b, s, 0)),
            pl.BlockSpec((1, s_blk, 1), lambda b, s: (b, s, 0)),
            pl.BlockSpec((1, 1, N), lambda b, s: (b, 0, 0)),
            pl.BlockSpec((1, 1, N), lambda b, s: (b, 0, 0)),
            pl.BlockSpec((1, 1, N), lambda b, s: (b, 0, 0)),
        ],
        out_specs=pl.BlockSpec((1, s_blk, 128), lambda b, s: (b, s, 0)),
        out_shape=jax.ShapeDtypeStruct((B, S, 128), I32),
    )(qxc, qyc, qzc, b3(bx), b3(by), b3(bz))


def _mlp1_body(x, w1, b1, w2, b2, w3, b3, out):
    h = jnp.maximum(jnp.dot(x[:, :], w1[:, :], preferred_element_type=F32) + b1[:, :], 0.0)
    h = jnp.maximum(jnp.dot(h, w2[:, :], preferred_element_type=F32) + b2[:, :], 0.0)
    h = jnp.maximum(jnp.dot(h, w3[:, :], preferred_element_type=F32) + b3[:, :], 0.0)
    out[:, :] = h


def _mlp1(x8, w1p, b1, w2, b2, w3, b3, blk):
    R = x8.shape[0]
    grid = (R // blk,)
    return pl.pallas_call(
        _mlp1_body,
        grid=grid,
        compiler_params=pltpu.CompilerParams(
            dimension_semantics=("parallel",)),
        in_specs=[
            pl.BlockSpec((blk, 8), lambda i: (i, 0)),
            pl.BlockSpec(w1p.shape, lambda i: (0, 0)),
            pl.BlockSpec(b1.shape, lambda i: (0, 0)),
            pl.BlockSpec(w2.shape, lambda i: (0, 0)),
            pl.BlockSpec(b2.shape, lambda i: (0, 0)),
            pl.BlockSpec(w3.shape, lambda i: (0, 0)),
            pl.BlockSpec(b3.shape, lambda i: (0, 0)),
        ],
        out_specs=pl.BlockSpec((blk, 128), lambda i: (i, 0)),
        out_shape=jax.ShapeDtypeStruct((R, 128), F32),
    )(x8, w1p, b1, w2, b2, w3, b3)


def _gmax_body(nsample, N, gi, h, out):
    SB = gi.shape[1]
    g = gi[0][:, :nsample]
    hb = h[0]
    iota = lax.broadcasted_iota(I32, (SB, N), 1)
    acc = jnp.full((SB, hb.shape[1]), -jnp.inf, dtype=F32)
    for k in range(nsample):
        oh = (g[:, k:k + 1] == iota).astype(F32)
        acc = jnp.maximum(acc, jnp.dot(oh, hb, preferred_element_type=F32))
    out[0] = acc


def _gather_max(gi, h3, nsample, s_blk):
    B, S, _ = gi.shape
    N, C = h3.shape[1], h3.shape[2]
    grid = (B, S // s_blk)
    return pl.pallas_call(
        functools.partial(_gmax_body, nsample, N),
        grid=grid,
        compiler_params=pltpu.CompilerParams(
            dimension_semantics=("parallel", "parallel")),
        in_specs=[
            pl.BlockSpec((1, s_blk, 128), lambda b, s: (b, s, 0)),
            pl.BlockSpec((1, N, C), lambda b, s: (b, 0, 0)),
        ],
        out_specs=pl.BlockSpec((1, s_blk, C), lambda b, s: (b, s, 0)),
        out_shape=jax.ShapeDtypeStruct((B, S, C), F32),
    )(gi, h3)


def _sa2_body(nsample, N, gi, x1p, f1, c8, w1a, w1b, b1, w2, b2, w3, b3, out):
    S = gi.shape[1]
    g = gi[0][:, :nsample]
    x1 = x1p[0]
    f1b = f1[0]
    ctr = c8[0]
    iota = lax.broadcasted_iota(I32, (S, N), 1)
    acc = jnp.full((S, w3.shape[1]), -jnp.inf, dtype=F32)
    for k in range(nsample):
        oh = (g[:, k:k + 1] == iota).astype(F32)
        gx = jnp.dot(oh, x1, preferred_element_type=F32) - ctr
        gf = jnp.dot(oh, f1b, preferred_element_type=F32)
        h = jnp.maximum(jnp.dot(gx, w1a[:, :], preferred_element_type=F32)
                        + jnp.dot(gf, w1b[:, :], preferred_element_type=F32)
                        + b1[:, :], 0.0)
        h = jnp.maximum(jnp.dot(h, w2[:, :], preferred_element_type=F32) + b2[:, :], 0.0)
        h = jnp.maximum(jnp.dot(h, w3[:, :], preferred_element_type=F32) + b3[:, :], 0.0)
        acc = jnp.maximum(acc, h)
    out[0] = acc


def _sa2(gi2, x1p8, f1, c8, w1a, w1b, b1, w2, b2, w3, b3, nsample):
    B, S, _ = gi2.shape
    N = x1p8.shape[1]
    Cout = w3.shape[1]
    return pl.pallas_call(
        functools.partial(_sa2_body, nsample, N),
        grid=(B,),
        compiler_params=pltpu.CompilerParams(
            dimension_semantics=("parallel",)),
        in_specs=[
            pl.BlockSpec((1, S, 128), lambda b: (b, 0, 0)),
            pl.BlockSpec((1, N, 8), lambda b: (b, 0, 0)),
            pl.BlockSpec((1, N, 128), lambda b: (b, 0, 0)),
            pl.BlockSpec((1, S, 8), lambda b: (b, 0, 0)),
            pl.BlockSpec(w1a.shape, lambda b: (0, 0)),
            pl.BlockSpec(w1b.shape, lambda b: (0, 0)),
            pl.BlockSpec(b1.shape, lambda b: (0, 0)),
            pl.BlockSpec(w2.shape, lambda b: (0, 0)),
            pl.BlockSpec(b2.shape, lambda b: (0, 0)),
            pl.BlockSpec(w3.shape, lambda b: (0, 0)),
            pl.BlockSpec(b3.shape, lambda b: (0, 0)),
        ],
        out_specs=pl.BlockSpec((1, S, Cout), lambda b: (b, 0, 0)),
        out_shape=jax.ShapeDtypeStruct((B, S, Cout), F32),
    )(gi2, x1p8, f1, c8, w1a, w1b, b1, w2, b2, w3, b3)


def _sa3fp1_body(c8, f2, w3a, w3b, b1, w2, b2, w3, b3, wfa, wfb, bf1, wf2, bf2, out):
    x = c8[0]
    f = f2[0]
    h = jnp.maximum(jnp.dot(x, w3a[:, :], preferred_element_type=F32)
                    + jnp.dot(f, w3b[:, :], preferred_element_type=F32) + b1[:, :], 0.0)
    h = jnp.maximum(jnp.dot(h, w2[:, :], preferred_element_type=F32) + b2[:, :], 0.0)
    h = jnp.maximum(jnp.dot(h, w3[:, :], preferred_element_type=F32) + b3[:, :], 0.0)
    f3 = jnp.max(h, axis=0, keepdims=True)
    u = jnp.maximum(jnp.dot(f3, wfa[:, :], preferred_element_type=F32)
                    + jnp.dot(f, wfb[:, :], preferred_element_type=F32) + bf1[:, :], 0.0)
    u = jnp.maximum(jnp.dot(u, wf2[:, :], preferred_element_type=F32) + bf2[:, :], 0.0)
    out[0] = u


def _sa3fp1(c8, f2, w3a, w3b, b1, w2, b2, w3, b3, wfa, wfb, bf1, wf2, bf2):
    B, S, _ = f2.shape
    Cout = wf2.shape[1]
    full = lambda a: pl.BlockSpec(a.shape, lambda b: tuple(0 for _ in a.shape))
    return pl.pallas_call(
        _sa3fp1_body,
        grid=(B,),
        compiler_params=pltpu.CompilerParams(
            dimension_semantics=("parallel",)),
        in_specs=[
            pl.BlockSpec((1, S, 8), lambda b: (b, 0, 0)),
            pl.BlockSpec((1, S, 256), lambda b: (b, 0, 0)),
            full(w3a), full(w3b), full(b1), full(w2), full(b2), full(w3), full(b3),
            full(wfa), full(wfb), full(bf1), full(wf2), full(bf2),
        ],
        out_specs=pl.BlockSpec((1, S, Cout), lambda b: (b, 0, 0)),
        out_shape=jax.ShapeDtypeStruct((B, S, Cout), F32),
    )(c8, f2, w3a, w3b, b1, w2, b2, w3, b3, wfa, wfb, bf1, wf2, bf2)


def _fp2_body(M, qx, qy, qz, bx, by, bz, f2u, f1, wa, wb, b1, w2, b2, out):
    S = qx.shape[1]
    dx = qx[0] - bx[0]
    dy = qy[0] - by[0]
    dz = qz[0] - bz[0]
    sqr = dx * dx + dy * dy + dz * dz
    iota = lax.broadcasted_iota(I32, (S, M), 1)
    fu = f2u[0]
    ws = []
    gs = []
    for _ in range(3):
        mv = jnp.min(sqr, axis=-1, keepdims=True)
        am = jnp.min(jnp.where(sqr == mv, iota, M), axis=-1, keepdims=True)
        oh = (iota == am).astype(F32)
        gs.append(jnp.dot(oh, fu, preferred_element_type=F32))
        ws.append(1.0 / jnp.maximum(mv, 1e-10))
        sqr = jnp.where(iota == am, jnp.float32(1e30), sqr)
    wsum = ws[0] + ws[1] + ws[2]
    interp = (ws[0] * gs[0] + ws[1] * gs[1] + ws[2] * gs[2]) / wsum
    h = jnp.maximum(jnp.dot(interp, wa[:, :], preferred_element_type=F32)
                    + jnp.dot(f1[0], wb[:, :], preferred_element_type=F32) + b1[:, :], 0.0)
    out[0] = jnp.maximum(jnp.dot(h, w2[:, :], preferred_element_type=F32) + b2[:, :], 0.0)


def _fp2(qxc, qyc, qzc, bx, by, bz, f2u, f1, wa, wb, b1, w2, b2):
    B, S, _ = qxc.shape
    M = bx.shape[1]
    Cout = w2.shape[1]
    full = lambda a: pl.BlockSpec(a.shape, lambda b: tuple(0 for _ in a.shape))
    return pl.pallas_call(
        functools.partial(_fp2_body, M),
        grid=(B,),
        compiler_params=pltpu.CompilerParams(
            dimension_semantics=("parallel",)),
        in_specs=[
            pl.BlockSpec((1, S, 1), lambda b: (b, 0, 0)),
            pl.BlockSpec((1, S, 1), lambda b: (b, 0, 0)),
            pl.BlockSpec((1, S, 1), lambda b: (b, 0, 0)),
            pl.BlockSpec((1, 1, M), lambda b: (b, 0, 0)),
            pl.BlockSpec((1, 1, M), lambda b: (b, 0, 0)),
            pl.BlockSpec((1, 1, M), lambda b: (b, 0, 0)),
            pl.BlockSpec((1, M, 256), lambda b: (b, 0, 0)),
            pl.BlockSpec((1, S, 128), lambda b: (b, 0, 0)),
            full(wa), full(wb), full(b1), full(w2), full(b2),
        ],
        out_specs=pl.BlockSpec((1, S, Cout), lambda b: (b, 0, 0)),
        out_shape=jax.ShapeDtypeStruct((B, S, Cout), F32),
    )(qxc, qyc, qzc, bx[:, None, :], by[:, None, :], bz[:, None, :],
      f2u, f1, wa, wb, b1, w2, b2)


def _sc_gather_max(table, idx, nsample):
    info = plsc.get_sparse_core_info()
    NC, NS = info.num_cores, info.num_subcores
    NW = NC * NS
    M = idx.shape[0]
    per_w = M // NW
    CH = 256
    n_ch = per_w // CH
    OUT_CH = CH // nsample
    mesh = plsc.VectorSubcoreMesh(core_axis_name="c", subcore_axis_name="s")

    @functools.partial(
        pl.kernel, mesh=mesh,
        out_type=jax.ShapeDtypeStruct((M // nsample, 128), F32),
        scratch_types=[
            pltpu.VMEM((CH,), I32),
            pltpu.VMEM((CH,), I32),
            pltpu.VMEM((CH, 128), F32),
            pltpu.VMEM((CH, 128), F32),
            pltpu.VMEM((OUT_CH, 128), F32),
            pltpu.SemaphoreType.DMA,
            pltpu.SemaphoreType.DMA,
        ],
    )
    def k(table_hbm, idx_hbm, out_hbm, idx0, idx1, rows0, rows1, out_v,
          sem0, sem1):
        wid = lax.axis_index("s") * NC + lax.axis_index("c")
        idx_v = (idx0, idx1)
        rows_v = (rows0, rows1)
        sems = (sem0, sem1)

        def fire(ch):
            base = pl.multiple_of(wid * per_w + ch * CH, CH)
            pltpu.sync_copy(idx_hbm.at[pl.ds(base, CH)], idx_v[ch % 2])
            return pltpu.async_copy(
                table_hbm.at[idx_v[ch % 2]], rows_v[ch % 2], sems[ch % 2])

        copies = {0: fire(0)}
        for ch in range(n_ch):
            if ch + 1 < n_ch:
                copies[ch + 1] = fire(ch + 1)
            copies[ch].wait()
            rv = rows_v[ch % 2]

            def grp(g, _):
                acc = [jnp.zeros((16,), F32)] * 8
                for r in range(nsample):
                    row = g * nsample + r
                    acc = [jnp.maximum(acc[v], rv[row, pl.ds(16 * v, 16)])
                           for v in range(8)]
                for v in range(8):
                    out_v[g, pl.ds(16 * v, 16)] = acc[v]
                return 0

            lax.fori_loop(0, OUT_CH, grp, 0)
            ob = pl.multiple_of((wid * per_w + ch * CH) // nsample, OUT_CH)
            pltpu.sync_copy(out_v, out_hbm.at[pl.ds(ob, OUT_CH)])

    return k(table, idx)


def _pad_rows(w, rows):
    return jnp.concatenate([w, jnp.zeros((rows - w.shape[0], w.shape[1]), F32)], axis=0)


def kernel(xyz, params):
    B, N, _ = xyz.shape
    xs = xyz[:, :, 0]
    ys = xyz[:, :, 1]
    zs = xyz[:, :, 2]

    cx1, cy1, cz1 = _fps(xs, ys, zs, 512)
    gi1 = _ball_query(0.1, 16, xs, ys, zs,
                      cx1[:, :, None], cy1[:, :, None], cz1[:, :, None],
                      flat=True, s_blk=128)
    w1, w2, w3 = params['sa1_w']
    b1, b2, b3 = params['sa1_b']
    x8 = jnp.concatenate([xyz, jnp.zeros((B, N, 5), F32)], axis=-1).reshape(B * N, 8)
    h_all = _mlp1(x8, _pad_rows(w1, 8), b1[None, :], w2, b2[None, :], w3, b3[None, :],
                  blk=4096)
    f1 = _sc_gather_max(h_all, gi1[:, :, :16].reshape(-1), 16).reshape(B, 512, 128)

    cx2, cy2, cz2 = _fps(cx1, cy1, cz1, 128)
    gi2 = _ball_query(0.2, 16, cx1, cy1, cz1,
                      cx2[:, :, None], cy2[:, :, None], cz2[:, :, None],
                      flat=False, s_blk=128)
    x1p8 = jnp.concatenate(
        [cx1[:, :, None], cy1[:, :, None], cz1[:, :, None],
         jnp.zeros((B, 512, 5), F32)], axis=-1)
    c8 = jnp.concatenate(
        [cx2[:, :, None], cy2[:, :, None], cz2[:, :, None],
         jnp.zeros((B, 128, 5), F32)], axis=-1)
    sw1, sw2, sw3 = params['sa2_w']
    sb1, sb2, sb3 = params['sa2_b']
    f2 = _sa2(gi2, x1p8, f1, c8,
              _pad_rows(sw1[:3], 8), sw1[3:], sb1[None, :],
              sw2, sb2[None, :], sw3, sb3[None, :], nsample=16)

    tw1, tw2, tw3 = params['sa3_w']
    tb1, tb2, tb3 = params['sa3_b']
    fw1, fw2 = params['fp1_w']
    fb1, fb2 = params['fp1_b']
    f2u = _sa3fp1(c8, f2,
                  _pad_rows(tw1[:3], 8), tw1[3:], tb1[None, :],
                  tw2, tb2[None, :], tw3, tb3[None, :],
                  fw1[:1024], fw1[1024:], fb1[None, :], fw2, fb2[None, :])

    pw1, pw2 = params['fp2_w']
    pb1, pb2 = params['fp2_b']
    out = _fp2(cx1[:, :, None], cy1[:, :, None], cz1[:, :, None],
               cx2, cy2, cz2, f2u, f1,
               pw1[:256], pw1[256:], pb1[None, :], pw2, pb2[None, :])
    return out
